# baseline (speedup 1.0000x reference)
import functools

import jax
import jax.numpy as jnp
from jax import lax
from jax.experimental import pallas as pl
from jax.experimental.pallas import tpu as pltpu
from jax.experimental.pallas import tpu_sc as plsc

_K = 8192
_D = 256
_N = 32768
_BN = 512
_NI = _N // _BN
_CK = _K
_NC = _K // _CK

_NW = 32
_BPW = _N // _NW
_CH = 128
_NCH = _BPW // _CH


def _dist_argmin_body(x_ref, w_ref, x2_ref, w2_ref, idx_ref, fii_ref):
    @pl.when(pl.program_id(0) == 0)
    def _():
        fii_ref[...] = lax.broadcasted_iota(
            jnp.int32, (1, _K), 1).astype(jnp.float32)

    xb = x_ref[...]
    sim = lax.dot_general(
        xb, w_ref[...], (((1,), (1,)), ((), ())),
        preferred_element_type=jnp.float32,
        precision=lax.Precision.DEFAULT)
    t = sim - (x2_ref[...] + w2_ref[...])
    m = jnp.max(t, axis=1, keepdims=True)
    fidx = jnp.min(jnp.where(t >= m, fii_ref[...], float(_K)), axis=1)
    idx_ref[...] = fidx.astype(jnp.int32)[:, None]


def _tc_argmin(xf, W16, x2h, w2h):
    return pl.pallas_call(
        _dist_argmin_body,
        grid=(_NI,),
        in_specs=[
            pl.BlockSpec((_BN, _D), lambda i: (i, 0)),
            pl.BlockSpec((_K, _D), lambda i: (0, 0)),
            pl.BlockSpec((_BN, 1), lambda i: (i, 0)),
            pl.BlockSpec((1, _K), lambda i: (0, 0)),
        ],
        out_specs=pl.BlockSpec((_BN, 1), lambda i: (i, 0)),
        out_shape=jax.ShapeDtypeStruct((_N, 1), jnp.int32),
        scratch_shapes=[pltpu.VMEM((1, _K), jnp.float32)],
    )(xf, W16, x2h, w2h)


@functools.cache
def _sc_gather_fn():
    @functools.partial(
        pl.kernel,
        mesh=plsc.VectorSubcoreMesh(core_axis_name="c", subcore_axis_name="s"),
        out_type=jax.ShapeDtypeStruct((_N, _D), jnp.float32),
        scratch_types=[
            pltpu.VMEM((_NCH, _CH), jnp.int32),
            pltpu.VMEM((_CH, _D), jnp.float32),
            pltpu.SemaphoreType.DMA,
        ],
    )
    def _sc_gather(w_hbm, idx_hbm, out_hbm, idx_v, rows_v, sem):
        wid = lax.axis_index("s") * 2 + lax.axis_index("c")
        pltpu.sync_copy(idx_hbm.at[pl.ds(wid * _NCH, _NCH)], idx_v)
        for c in range(_NCH):
            pltpu.async_copy(w_hbm.at[idx_v.at[c]], rows_v, sem).wait()
            pltpu.sync_copy(rows_v, out_hbm.at[pl.ds(wid * _BPW + c * _CH, _CH)])

    return _sc_gather


def kernel(x, W):
    xf = x.reshape(-1, _D)
    x2h = 0.5 * jnp.sum(xf ** 2, axis=1, keepdims=True)
    w2h = (0.5 * jnp.sum(W ** 2, axis=1)).reshape(1, _K)
    idx = _tc_argmin(xf, W, x2h, w2h)
    q = _sc_gather_fn()(W, idx.reshape(_NW * _NCH, _CH))
    return q.reshape(x.shape), idx


# speedup vs baseline: 1.1780x; 1.1780x over previous
import functools

import jax
import jax.numpy as jnp
from jax import lax
from jax.experimental import pallas as pl
from jax.experimental.pallas import tpu as pltpu
from jax.experimental.pallas import tpu_sc as plsc

_K = 8192
_D = 256
_N = 32768
_BN = 512
_NI = _N // _BN
_CK = _K
_NC = _K // _CK

_NW = 32
_BPW = _N // _NW
_CH = 128
_NCH = _BPW // _CH


def _dist_argmin_body(x_ref, w_ref, x2_ref, w2_ref, idx_ref, fii_ref):
    @pl.when(pl.program_id(0) == 0)
    def _():
        fii_ref[...] = lax.broadcasted_iota(
            jnp.int32, (1, _K), 1).astype(jnp.float32)

    xb = x_ref[...]
    sim = lax.dot_general(
        xb, w_ref[...], (((1,), (1,)), ((), ())),
        preferred_element_type=jnp.float32,
        precision=lax.Precision.DEFAULT)
    d = (x2_ref[...] + w2_ref[...]) - 2.0 * sim
    m = jnp.min(d, axis=1, keepdims=True)
    fidx = jnp.min(jnp.where(d <= m, fii_ref[...], float(_K)), axis=1)
    idx_ref[...] = fidx.astype(jnp.int32)[:, None]


def _tc_argmin(xf, W16, x2h, w2h):
    return pl.pallas_call(
        _dist_argmin_body,
        grid=(_NI,),
        in_specs=[
            pl.BlockSpec((_BN, _D), lambda i: (i, 0)),
            pl.BlockSpec((_K, _D), lambda i: (0, 0)),
            pl.BlockSpec((_BN, 1), lambda i: (i, 0)),
            pl.BlockSpec((1, _K), lambda i: (0, 0)),
        ],
        out_specs=pl.BlockSpec((_BN, 1), lambda i: (i, 0)),
        out_shape=jax.ShapeDtypeStruct((_N, 1), jnp.int32),
        scratch_shapes=[pltpu.VMEM((1, _K), jnp.float32)],
    )(xf, W16, x2h, w2h)


@functools.cache
def _sc_gather_fn():
    @functools.partial(
        pl.kernel,
        mesh=plsc.VectorSubcoreMesh(core_axis_name="c", subcore_axis_name="s"),
        out_type=jax.ShapeDtypeStruct((_N, _D), jnp.float32),
        scratch_types=[
            pltpu.VMEM((_NCH, _CH), jnp.int32),
            pltpu.VMEM((_CH, _D), jnp.float32),
            pltpu.SemaphoreType.DMA,
        ],
    )
    def _sc_gather(w_hbm, idx_hbm, out_hbm, idx_v, rows_v, sem):
        wid = lax.axis_index("s") * 2 + lax.axis_index("c")
        pltpu.sync_copy(idx_hbm.at[pl.ds(wid * _NCH, _NCH)], idx_v)
        for c in range(_NCH):
            pltpu.async_copy(w_hbm.at[idx_v.at[c]], rows_v, sem).wait()
            pltpu.sync_copy(rows_v, out_hbm.at[pl.ds(wid * _BPW + c * _CH, _CH)])

    return _sc_gather


def kernel(x, W):
    xf = x.reshape(-1, _D)
    x2 = jnp.sum(xf ** 2, axis=1, keepdims=True)
    w2 = jnp.sum(W ** 2, axis=1).reshape(1, _K)
    idx = _tc_argmin(xf, W, x2, w2)
    q = _sc_gather_fn()(W, idx.reshape(_NW * _NCH, _CH))
    return q.reshape(x.shape), idx


# speedup vs baseline: 1.1799x; 1.0016x over previous
import functools

import jax
import jax.numpy as jnp
from jax import lax
from jax.experimental import pallas as pl
from jax.experimental.pallas import tpu as pltpu
from jax.experimental.pallas import tpu_sc as plsc

_K = 8192
_D = 256
_N = 32768
_BN = 512
_NI = _N // _BN

_NW = 32
_BPW = _N // _NW
_CH = 128
_NCH = _BPW // _CH


def _dist_argmin_body(x_ref, w_ref, x2_ref, w2_ref, idx_ref, fii_ref):
    @pl.when(pl.program_id(0) == 0)
    def _():
        fii_ref[...] = lax.broadcasted_iota(
            jnp.int32, (1, _K), 1).astype(jnp.float32)

    xb = x_ref[...]
    sim = lax.dot_general(
        xb, w_ref[...], (((1,), (1,)), ((), ())),
        preferred_element_type=jnp.float32,
        precision=lax.Precision.DEFAULT)
    d = (x2_ref[...] + w2_ref[...]) - 2.0 * sim
    m = jnp.min(d, axis=1, keepdims=True)
    fidx = jnp.min(jnp.where(d <= m, fii_ref[...], float(_K)), axis=1)
    idx_ref[...] = fidx.astype(jnp.int32)[:, None]


def _tc_argmin(xf, W, x2, w2):
    return pl.pallas_call(
        _dist_argmin_body,
        grid=(_NI,),
        in_specs=[
            pl.BlockSpec((_BN, _D), lambda i: (i, 0)),
            pl.BlockSpec((_K, _D), lambda i: (0, 0)),
            pl.BlockSpec((_BN, 1), lambda i: (i, 0)),
            pl.BlockSpec((1, _K), lambda i: (0, 0)),
        ],
        out_specs=pl.BlockSpec((_BN, 1), lambda i: (i, 0)),
        out_shape=jax.ShapeDtypeStruct((_N, 1), jnp.int32),
        scratch_shapes=[pltpu.VMEM((1, _K), jnp.float32)],
    )(xf, W, x2, w2)


@functools.cache
def _sc_gather_fn():
    @functools.partial(
        pl.kernel,
        mesh=plsc.VectorSubcoreMesh(core_axis_name="c", subcore_axis_name="s"),
        out_type=jax.ShapeDtypeStruct((_N, _D), jnp.float32),
        scratch_types=[
            pltpu.VMEM((_NCH, _CH), jnp.int32),
            pltpu.VMEM((_CH, _D), jnp.float32),
            pltpu.SemaphoreType.DMA,
        ],
    )
    def _sc_gather(w_hbm, idx_hbm, out_hbm, idx_v, rows_v, sem):
        wid = lax.axis_index("s") * 2 + lax.axis_index("c")
        pltpu.sync_copy(idx_hbm.at[pl.ds(wid * _NCH, _NCH)], idx_v)
        for c in range(_NCH):
            pltpu.async_copy(w_hbm.at[idx_v.at[c]], rows_v, sem).wait()
            pltpu.sync_copy(rows_v, out_hbm.at[pl.ds(wid * _BPW + c * _CH, _CH)])

    return _sc_gather


def kernel(x, W):
    xf = x.reshape(-1, _D)
    x2 = jnp.sum(xf ** 2, axis=1, keepdims=True)
    w2 = jnp.sum(W ** 2, axis=1).reshape(1, _K)
    idx = _tc_argmin(xf, W, x2, w2)
    q = _sc_gather_fn()(W, idx.reshape(_NW * _NCH, _CH))
    return q.reshape(x.shape), idx
